# baseline (speedup 1.0000x reference)
import functools

import jax
import jax.numpy as jnp
from jax import lax
from jax.experimental import pallas as pl
from jax.experimental.pallas import tpu as pltpu
from jax.experimental.pallas import tpu_sc as plsc

ROWS = 64
N = 32768
L = 16
BLK = 128
NBLK = N // BLK
NBC = NBLK // L
GMAX = 96
NGATH = 2
CV = 4096
T_NEWTON = 10
NEG = -3e38

_NC = 2
_NS = 16
NW = _NC * _NS
ROWS_PER = ROWS // NW

TCR = 16


def _tc_blockmax(x_ref, bm_ref):
    x = x_ref[...]
    bm_ref[...] = jnp.max(x.reshape(TCR, NBLK, BLK), axis=-1)


def _tc_out(x_ref, tau_ref, o_ref):
    t = jnp.max(tau_ref[...], axis=-1, keepdims=True)
    o_ref[...] = jnp.maximum(x_ref[...] - t, jnp.float32(0))


def _sc_taus(xblk_hbm, bm_hbm, taus_hbm, bmrow_v, bidx0_v, bidx1_v,
             blk0_v, blk1_v, vals_v, row_v, tau_stage, sem):
    wid = lax.axis_index("s") * _NC + lax.axis_index("c")
    for j in range(ROWS_PER):
        r = wid * ROWS_PER + j
        pltpu.sync_copy(bm_hbm.at[r], bmrow_v)

        c = [bmrow_v[pl.ds(i * L, L)] for i in range(NBC)]
        m = c[0]
        for i in range(1, NBC):
            m = jnp.maximum(m, c[i])
        mx = jnp.max(m)
        thr = jnp.broadcast_to(mx, (L,)) - 1.0

        base_id = jnp.broadcast_to(r * NBLK, (L,))
        for i in range(GMAX // L):
            bidx0_v[pl.ds(i * L, L)] = base_id
            bidx1_v[pl.ds(i * L, L)] = base_id
        nb = jnp.int32(0)
        for i in range(NBC):
            msk = c[i] > thr
            mi = msk.astype(jnp.int32)
            pos = plsc.cumsum(mi) - 1 + nb
            gid = base_id + i * L + lax.iota(jnp.int32, L)
            okm0 = jnp.logical_and(msk, pos < GMAX)
            plsc.store_scatter(bidx0_v, [pos], gid, mask=okm0)
            okm1 = jnp.logical_and(msk, jnp.logical_and(pos >= GMAX,
                                                        pos < 2 * GMAX))
            plsc.store_scatter(bidx1_v, [pos - GMAX], gid, mask=okm1)
            nb = nb + plsc.all_reduce_population_count(msk)[0]
        overflow = nb > 2 * GMAX

        pltpu.async_copy(xblk_hbm.at[bidx0_v], blk0_v, sem).wait()
        pltpu.async_copy(xblk_hbm.at[bidx1_v], blk1_v, sem).wait()

        def shrink_blk(blk_ref, lo, hi, off2):
            def sb(bi, o2):
                o = o2
                for k in range(BLK // L):
                    v = blk_ref[bi, pl.ds(k * L, L)]
                    msk = v > thr
                    cnt = plsc.all_reduce_population_count(msk)[0]
                    plsc.store_compressed(vals_v.at[pl.ds(o, L)], v, mask=msk)
                    o = o + jnp.where(o < CV - 2 * L, cnt, 0)
                return o
            return lax.fori_loop(lo, hi, sb, off2)

        k1 = shrink_blk(blk0_v, 0, jnp.minimum(nb, GMAX), jnp.int32(0))
        k1 = shrink_blk(blk1_v, 0, jnp.maximum(nb - GMAX, 0), k1)
        overflow = jnp.logical_or(overflow, k1 >= CV - 2 * L)
        vals_v[pl.ds(k1, L)] = jnp.full((L,), NEG, jnp.float32)
        nv = (k1 + (L - 1)) >> 4

        @pl.when(overflow)
        def _():
            pltpu.sync_copy(xblk_hbm.at[pl.ds(r * NBLK, NBLK)], row_v)

        def _sk_pass(read_chunk, n_chunks, tau):
            def b(i, sk):
                sv, kv = sk
                v = read_chunk(i)
                msk = v > tau
                sv = sv + jnp.where(msk, v, jnp.float32(0))
                kv = kv + msk.astype(jnp.int32)
                return (sv, kv)
            return lax.fori_loop(
                0, n_chunks, b,
                (jnp.zeros((L,), jnp.float32), jnp.zeros((L,), jnp.int32)))

        def newton_body(t, tau):
            sv, kv = lax.cond(
                overflow,
                lambda tt: _sk_pass(
                    lambda i: row_v[i >> 3, pl.ds((i & 7) * L, L)],
                    N // L, tt),
                lambda tt: _sk_pass(
                    lambda i: vals_v[pl.ds(i * L, L)], nv, tt),
                tau)
            s = jnp.sum(sv)
            kf = jnp.sum(kv.astype(jnp.float32))
            kfv = jnp.maximum(jnp.broadcast_to(kf, (L,)), 1.0)
            tau_new = (jnp.broadcast_to(s, (L,)) - 1.0) / kfv
            return jnp.maximum(tau, tau_new)
        tau = lax.fori_loop(0, T_NEWTON, newton_body, thr)

        for i in range(BLK // L):
            tau_stage[pl.ds(i * L, L)] = tau
        pltpu.sync_copy(tau_stage, taus_hbm.at[r])


@jax.jit
def kernel(input):
    bm = pl.pallas_call(
        _tc_blockmax,
        grid=(ROWS // TCR,),
        in_specs=[pl.BlockSpec((TCR, N), lambda i: (i, 0))],
        out_specs=pl.BlockSpec((TCR, NBLK), lambda i: (i, 0)),
        out_shape=jax.ShapeDtypeStruct((ROWS, NBLK), jnp.float32),
    )(input)

    xblk = input.reshape(ROWS * NBLK, BLK)
    mesh = plsc.VectorSubcoreMesh(core_axis_name="c", subcore_axis_name="s")
    taus = pl.kernel(
        _sc_taus,
        out_type=jax.ShapeDtypeStruct((ROWS, BLK), jnp.float32),
        mesh=mesh,
        scratch_types=[
            pltpu.VMEM((NBLK,), jnp.float32),
            pltpu.VMEM((GMAX,), jnp.int32),
            pltpu.VMEM((GMAX,), jnp.int32),
            pltpu.VMEM((GMAX, BLK), jnp.float32),
            pltpu.VMEM((GMAX, BLK), jnp.float32),
            pltpu.VMEM((CV,), jnp.float32),
            pltpu.VMEM((NBLK, BLK), jnp.float32),
            pltpu.VMEM((BLK,), jnp.float32),
            pltpu.SemaphoreType.DMA,
        ],
        compiler_params=pltpu.CompilerParams(needs_layout_passes=False),
    )(xblk, bm)

    out = pl.pallas_call(
        _tc_out,
        grid=(ROWS // TCR,),
        in_specs=[pl.BlockSpec((TCR, N), lambda i: (i, 0)),
                  pl.BlockSpec((TCR, BLK), lambda i: (i, 0))],
        out_specs=pl.BlockSpec((TCR, N), lambda i: (i, 0)),
        out_shape=jax.ShapeDtypeStruct((ROWS, N), jnp.float32),
    )(input, taus)
    return out


# speedup vs baseline: 1.6654x; 1.6654x over previous
import functools

import jax
import jax.numpy as jnp
from jax import lax
from jax.experimental import pallas as pl
from jax.experimental.pallas import tpu as pltpu
from jax.experimental.pallas import tpu_sc as plsc

ROWS = 64
N = 32768
L = 16
NB = N // L
U = 8
NG = NB // U
T_NEWTON = 10
C = 24576
GPI = 4
NEG = -3e38

_NC = 2
_NS = 16
NW = _NC * _NS
ROWS_PER = ROWS // NW


def _tree_max8(c):
    t01 = jnp.maximum(c[0], c[1])
    t23 = jnp.maximum(c[2], c[3])
    t45 = jnp.maximum(c[4], c[5])
    t67 = jnp.maximum(c[6], c[7])
    return jnp.maximum(jnp.maximum(t01, t23), jnp.maximum(t45, t67))


def _fmc_start(row_v):
    g0 = [row_v[pl.ds(j * L, L)] for j in range(U)]
    m0 = _tree_max8(g0)
    w = jnp.broadcast_to(jnp.max(m0), (L,)) - 1.0
    return (m0, w, w, jnp.int32(0))


def _fmc_range(row_v, cand_v, lo, hi, carry):
    def body(it, carry):
        m, t0, t1, off = carry
        m_new = m
        for gg in range(GPI):
            base = (it * GPI + gg) * (U * L)
            c = [row_v[pl.ds(base + j * L, L)] for j in range(U)]
            for j in range(0, U, 2):
                msk = jnp.logical_or(c[j] > t0, c[j + 1] > t0)
                cnt = plsc.all_reduce_population_count(msk)[0]
                cand_v[pl.ds(off, L)] = c[j]
                cand_v[pl.ds(off + L, L)] = c[j + 1]
                off = off + jnp.where(
                    jnp.logical_and(cnt > 0, off < C - 4 * L),
                    jnp.int32(2 * L), jnp.int32(0))
            m_new = jnp.maximum(m_new, _tree_max8(c))
        nt = jnp.broadcast_to(jnp.max(m_new), (L,)) - 1.0
        return (m_new, t1, nt, off)

    return lax.fori_loop(lo, hi, body, carry)


def _shrink(cand_v, vals_v, nb_c, thr):
    def body(i, off2):
        v = cand_v[pl.ds(i * L, L)]
        msk = v > thr
        cnt = plsc.all_reduce_population_count(msk)[0]
        plsc.store_compressed(vals_v.at[pl.ds(off2, L)], v, mask=msk)
        return off2 + cnt
    k1 = lax.fori_loop(0, nb_c, body, jnp.int32(0))
    vals_v[pl.ds(k1, L)] = jnp.full((L,), NEG, jnp.float32)
    return k1


def _row_tau(row_v, cand_v, vals_v, carry):
    m, _, _, off = carry
    mx = jnp.max(m)
    thr = jnp.broadcast_to(mx, (L,)) - 1.0
    overflow = off >= C - 2 * L

    k1 = _shrink(cand_v, vals_v, off >> 4, thr)
    nv = (k1 + (L - 1)) >> 4

    def _sk_pass(ref, n_chunks, tau):
        def b(i, sk):
            sv, kv = sk
            v = ref[pl.ds(i * L, L)]
            msk = v > tau
            sv = sv + jnp.where(msk, v, jnp.float32(0))
            kv = kv + msk.astype(jnp.int32)
            return (sv, kv)
        return lax.fori_loop(
            0, n_chunks, b,
            (jnp.zeros((L,), jnp.float32), jnp.zeros((L,), jnp.int32)))

    def newton_body(t, tau):
        sv, kv = lax.cond(
            overflow,
            lambda tt: _sk_pass(row_v, NB, tt),
            lambda tt: _sk_pass(vals_v, nv, tt),
            tau)
        s = jnp.sum(sv)
        kf = jnp.sum(kv.astype(jnp.float32))
        kfv = jnp.maximum(jnp.broadcast_to(kf, (L,)), 1.0)
        tau_new = (jnp.broadcast_to(s, (L,)) - 1.0) / kfv
        return jnp.maximum(tau, tau_new)
    return lax.fori_loop(0, T_NEWTON, newton_body, thr)


def _out_half(row_v, tau, half):
    def out_body(g):
        base = g * (U * L)
        for j in range(U):
            sl = pl.ds(base + j * L, L)
            row_v[sl] = jnp.maximum(row_v[sl] - tau, jnp.float32(0))
    plsc.parallel_loop(half * (NG // 2), (half + 1) * (NG // 2), 1,
                       unroll=2)(out_body)


HN = N // 2
NIT = NG // GPI


def _body(x_hbm, out_hbm, row_a, row_b, cand_v, vals_v, s0, s1, s2, s3):
    wid = lax.axis_index("s") * _NC + lax.axis_index("c")
    r0 = wid * ROWS_PER
    r1 = r0 + 1
    ia0 = pltpu.async_copy(x_hbm.at[r0, pl.ds(0, HN)],
                           row_a.at[pl.ds(0, HN)], s0)
    ia1 = pltpu.async_copy(x_hbm.at[r0, pl.ds(HN, HN)],
                           row_a.at[pl.ds(HN, HN)], s1)
    ib = pltpu.async_copy(x_hbm.at[r1], row_b, s2)

    ia0.wait()
    carry = _fmc_range(row_a, cand_v, 0, NIT // 2, _fmc_start(row_a))
    ia1.wait()
    carry = _fmc_range(row_a, cand_v, NIT // 2, NIT, carry)
    tau_a = _row_tau(row_a, cand_v, vals_v, carry)
    _out_half(row_a, tau_a, 0)
    oa0 = pltpu.async_copy(row_a.at[pl.ds(0, HN)],
                           out_hbm.at[r0, pl.ds(0, HN)], s0)
    _out_half(row_a, tau_a, 1)
    oa1 = pltpu.async_copy(row_a.at[pl.ds(HN, HN)],
                           out_hbm.at[r0, pl.ds(HN, HN)], s1)

    ib.wait()
    carry = _fmc_range(row_b, cand_v, 0, NIT, _fmc_start(row_b))
    tau_b = _row_tau(row_b, cand_v, vals_v, carry)
    _out_half(row_b, tau_b, 0)
    ob0 = pltpu.async_copy(row_b.at[pl.ds(0, HN)],
                           out_hbm.at[r1, pl.ds(0, HN)], s2)
    _out_half(row_b, tau_b, 1)
    ob1 = pltpu.async_copy(row_b.at[pl.ds(HN, HN)],
                           out_hbm.at[r1, pl.ds(HN, HN)], s3)
    oa0.wait()
    oa1.wait()
    ob0.wait()
    ob1.wait()


@jax.jit
def kernel(input):
    mesh = plsc.VectorSubcoreMesh(core_axis_name="c", subcore_axis_name="s")
    f = pl.kernel(
        _body,
        out_type=jax.ShapeDtypeStruct((ROWS, N), jnp.float32),
        mesh=mesh,
        scratch_types=[
            pltpu.VMEM((N,), jnp.float32),
            pltpu.VMEM((N,), jnp.float32),
            pltpu.VMEM((C,), jnp.float32),
            pltpu.VMEM((C,), jnp.float32),
            pltpu.SemaphoreType.DMA,
            pltpu.SemaphoreType.DMA,
            pltpu.SemaphoreType.DMA,
            pltpu.SemaphoreType.DMA,
        ],
        compiler_params=pltpu.CompilerParams(needs_layout_passes=False),
    )
    return f(input)
